# baseline (speedup 1.0000x reference)
import functools

import jax
import jax.numpy as jnp
from jax import lax
from jax.experimental import pallas as pl
from jax.experimental.pallas import tpu as pltpu
from jax.experimental.pallas import tpu_sc as plsc

N = 10000
E = 320000
D = 128

NC = 2
NS = 16
NW = NC * NS
K = 80
NB = 3
D2 = D // NC
NCH_A = E // (NS * K)
R0 = 624
RL = N - (NS - 1) * R0
ZR = 104
DW = 16

f32 = jnp.float32


def _zero_shared(sid, zrow_v, sh):
  lo = sid * R0
  for r in range(R0 // ZR):
    pltpu.sync_copy(zrow_v, sh.at[pl.ds(lo + r * ZR, ZR)])

  @pl.when(sid == NS - 1)
  def _():
    pltpu.sync_copy(zrow_v.at[pl.ds(0, RL - R0)],
                    sh.at[pl.ds(N - (RL - R0), RL - R0)])


def _writeback(core, sid, sh, out_hbm):
  @pl.when(sid < NS - 1)
  def _():
    pltpu.sync_copy(sh.at[pl.ds(sid * R0, R0)],
                    out_hbm.at[core, pl.ds(sid * R0, R0)])

  @pl.when(sid == NS - 1)
  def _():
    pltpu.sync_copy(sh.at[pl.ds((NS - 1) * R0, RL)],
                    out_hbm.at[core, pl.ds((NS - 1) * R0, RL)])


def _make_agg(with_deg, interleaved):
  mesh = plsc.VectorSubcoreMesh(core_axis_name="c", subcore_axis_name="s")
  dw = D2
  nch = NCH_A

  out_type = [jax.ShapeDtypeStruct((NC, N, dw), f32)]
  scratch = (
      [pltpu.VMEM((nch, K), jnp.int32),
       pltpu.VMEM((nch, K), jnp.int32)]
      + [pltpu.VMEM((K, dw), f32)] * NB
      + [pltpu.VMEM((ZR, dw), f32)]
      + [pltpu.VMEM_SHARED((N, dw), f32)]
      + [pltpu.SemaphoreType.DMA] * (2 * NB)
  )
  if with_deg:
    out_type.append(jax.ShapeDtypeStruct((NC, N, DW), f32))
    scratch += [
        pltpu.VMEM((K, DW), f32),
        pltpu.VMEM((ZR, DW), f32),
        pltpu.VMEM_SHARED((N, DW), f32),
    ]

  def body(feat_hbm, ei_hbm, *rest):
    rest = list(rest)
    agg_hbm = rest.pop(0)
    deg_hbm = rest.pop(0) if with_deg else None
    srcs_v, dsts_v = rest.pop(0), rest.pop(0)
    rb = [rest.pop(0) for _ in range(NB)]
    zrow_v = rest.pop(0)
    agg_sh = rest.pop(0)
    gsems = [rest.pop(0) for _ in range(NB)]
    ssems = [rest.pop(0) for _ in range(NB)]
    if with_deg:
      ones_v, zdeg_v, deg_sh = rest
    core = lax.axis_index("c")
    sid = lax.axis_index("s")

    def fill(i, _):
      for j in range(dw // 16):
        zrow_v[i, pl.ds(j * 16, 16)] = jnp.zeros((16,), f32)
      return 0

    lax.fori_loop(0, ZR, fill, 0)
    _zero_shared(sid, zrow_v, agg_sh)
    if with_deg:
      def dfill(i, _):
        zdeg_v[i, :] = jnp.zeros((16,), f32)
        return 0

      lax.fori_loop(0, ZR, dfill, 0)

      def ofill(i, _):
        ones_v[i, :] = jnp.full((16,), 0.5, f32)
        return 0

      lax.fori_loop(0, K, ofill, 0)
      _zero_shared(sid, zdeg_v, deg_sh)
    pltpu.sync_copy(ei_hbm.at[0, sid], srcs_v)
    pltpu.sync_copy(ei_hbm.at[1, sid], dsts_v)
    if interleaved:
      def xform(i, _):
        for j in range(K // 16):
          sl = pl.ds(j * 16, 16)
          srcs_v[i, sl] = srcs_v[i, sl] * 2 + core
        return 0

      lax.fori_loop(0, nch, xform, 0)
    plsc.subcore_barrier()

    feat = feat_hbm if interleaved else feat_hbm.at[core]

    for b in range(NB - 1):
      pltpu.async_copy(feat.at[srcs_v.at[b]], rb[b], gsems[b])

    def ring(i, _):
      for b in range(NB):
        c = i * NB + b
        nxt = (b + NB - 1) % NB
        pltpu.make_async_copy(feat.at[srcs_v.at[c]], rb[b], gsems[b]).wait()

        @pl.when(c + NB - 1 < nch)
        def _():
          @pl.when(c >= 1)
          def _():
            pltpu.make_async_copy(rb[nxt], agg_sh.at[dsts_v.at[c]],
                                  ssems[nxt]).wait()
          pltpu.async_copy(feat.at[srcs_v.at[c + NB - 1]], rb[nxt],
                           gsems[nxt])

        pltpu.async_copy(rb[b], agg_sh.at[dsts_v.at[c]], ssems[b], add=True)
        if with_deg:
          pltpu.sync_copy(ones_v, deg_sh.at[dsts_v.at[c]], add=True)
      return 0

    lax.fori_loop(0, nch // NB, ring, 0)
    for c in range(nch - nch % NB, nch):
      b = c % NB
      pltpu.make_async_copy(feat.at[srcs_v.at[c]], rb[b], gsems[b]).wait()
      pltpu.async_copy(rb[b], agg_sh.at[dsts_v.at[c]], ssems[b], add=True)
      if with_deg:
        pltpu.sync_copy(ones_v, deg_sh.at[dsts_v.at[c]], add=True)
    for c in range(nch - NB, nch):
      b = c % NB
      pltpu.make_async_copy(rb[b], agg_sh.at[dsts_v.at[0]], ssems[b]).wait()
    plsc.subcore_barrier()
    _writeback(core, sid, agg_sh, agg_hbm)
    if with_deg:
      _writeback(core, sid, deg_sh, deg_hbm)

  return functools.partial(
      pl.kernel, mesh=mesh, out_type=out_type, scratch_types=scratch,
      compiler_params=pltpu.CompilerParams(
          use_tc_tiling_on_sc=False, skip_device_barrier=True))(body)


_make_agg = functools.lru_cache(maxsize=None)(_make_agg)

BN = 2000


def _make_combine(relu, logsm, split_x, split_out, agg_full):
  def body(agg_ref, deg_ref, x_ref, wn_ref, ws_ref, b_ref, out_ref):
    if agg_full:
      agg = agg_ref[0] + agg_ref[1]
    else:
      agg = jnp.concatenate([agg_ref[0], agg_ref[1]], axis=1)
    deg = deg_ref[0][:, 0:1] + deg_ref[1][:, 0:1]
    mean = agg / jnp.maximum(deg, 1.0)
    if split_x:
      xv = jnp.concatenate([x_ref[0], x_ref[1]], axis=1)
    else:
      xv = x_ref[...]
    h = (jnp.dot(mean, wn_ref[...], preferred_element_type=f32,
                 precision=lax.Precision.HIGHEST)
         + jnp.dot(xv, ws_ref[...], preferred_element_type=f32,
                   precision=lax.Precision.HIGHEST)
         + b_ref[...])
    if relu:
      h = jnp.maximum(h, 0.0)
    if logsm:
      m = jnp.max(h, axis=1, keepdims=True)
      h = h - (jnp.log(jnp.sum(jnp.exp(h - m), axis=1, keepdims=True)) + m)
    if split_out:
      out_ref[0] = h[:, :D2]
      out_ref[1] = h[:, D2:]
    else:
      out_ref[...] = h

  x_spec = (pl.BlockSpec((2, BN, D2), lambda i: (0, i, 0)) if split_x
            else pl.BlockSpec((BN, D), lambda i: (i, 0)))
  if split_out:
    out_spec = pl.BlockSpec((2, BN, D2), lambda i: (0, i, 0))
    out_shape = jax.ShapeDtypeStruct((NC, N, D2), f32)
  else:
    out_spec = pl.BlockSpec((BN, D), lambda i: (i, 0))
    out_shape = jax.ShapeDtypeStruct((N, D), f32)

  return pl.pallas_call(
      body,
      grid=(N // BN,),
      in_specs=[
          pl.BlockSpec((2, BN, D if agg_full else D2), lambda i: (0, i, 0)),
          pl.BlockSpec((2, BN, DW), lambda i: (0, i, 0)),
          x_spec,
          pl.BlockSpec((D, D), lambda i: (0, 0)),
          pl.BlockSpec((D, D), lambda i: (0, 0)),
          pl.BlockSpec((1, D), lambda i: (0, 0)),
      ],
      out_specs=out_spec,
      out_shape=out_shape,
  )


_combine1 = _make_combine(relu=True, logsm=False, split_x=False,
                          split_out=True, agg_full=False)
_combine2 = _make_combine(relu=False, logsm=True, split_x=True,
                          split_out=False, agg_full=False)


def kernel(x, edge_index, W1_neigh, W1_self, b1, W2_neigh, W2_self, b2):
  ei4 = edge_index.reshape(2, NS, NCH_A, K)
  xi = x.reshape(NC * N, D2)
  agg1, degp = _make_agg(True, True)(xi, ei4)
  h2 = _combine1(agg1, degp, x, W1_neigh, W1_self, b1.reshape(1, D))
  (agg2,) = _make_agg(False, False)(h2, ei4)
  return _combine2(agg2, degp, h2, W2_neigh, W2_self, b2.reshape(1, D))


# speedup vs baseline: 1.1268x; 1.1268x over previous
import functools

import jax
import jax.numpy as jnp
from jax import lax
from jax.experimental import pallas as pl
from jax.experimental.pallas import tpu as pltpu
from jax.experimental.pallas import tpu_sc as plsc

N = 10000
E = 320000
D = 128

NC = 2
NS = 16
NW = NC * NS
K = 128
NB = 3
D2 = D // NC
EPW = E // NS
NCH_A = EPW // K
KT = EPW - NCH_A * K
R0 = 624
RL = N - (NS - 1) * R0
ZR = 104
DW = 16

f32 = jnp.float32


def _zero_shared(sid, zrow_v, sh):
  lo = sid * R0
  for r in range(R0 // ZR):
    pltpu.sync_copy(zrow_v, sh.at[pl.ds(lo + r * ZR, ZR)])

  @pl.when(sid == NS - 1)
  def _():
    pltpu.sync_copy(zrow_v.at[pl.ds(0, RL - R0)],
                    sh.at[pl.ds(N - (RL - R0), RL - R0)])


def _writeback(core, sid, sh, out_hbm):
  @pl.when(sid < NS - 1)
  def _():
    pltpu.sync_copy(sh.at[pl.ds(sid * R0, R0)],
                    out_hbm.at[core, pl.ds(sid * R0, R0)])

  @pl.when(sid == NS - 1)
  def _():
    pltpu.sync_copy(sh.at[pl.ds((NS - 1) * R0, RL)],
                    out_hbm.at[core, pl.ds((NS - 1) * R0, RL)])


def _make_agg(with_deg, interleaved):
  mesh = plsc.VectorSubcoreMesh(core_axis_name="c", subcore_axis_name="s")
  dw = D2
  nch = NCH_A

  out_type = [jax.ShapeDtypeStruct((NC, N, dw), f32)]
  scratch = (
      [pltpu.VMEM((EPW,), jnp.int32),
       pltpu.VMEM((EPW,), jnp.int32)]
      + [pltpu.VMEM((K, dw), f32)] * NB
      + [pltpu.VMEM((ZR, dw), f32)]
      + [pltpu.VMEM_SHARED((N, dw), f32)]
      + [pltpu.SemaphoreType.DMA] * (2 * NB)
  )
  if with_deg:
    out_type.append(jax.ShapeDtypeStruct((NC, N, DW), f32))
    scratch += [
        pltpu.VMEM((K, DW), f32),
        pltpu.VMEM((ZR, DW), f32),
        pltpu.VMEM_SHARED((N, DW), f32),
    ]

  def body(feat_hbm, ei_hbm, *rest):
    rest = list(rest)
    agg_hbm = rest.pop(0)
    deg_hbm = rest.pop(0) if with_deg else None
    srcs_v, dsts_v = rest.pop(0), rest.pop(0)
    rb = [rest.pop(0) for _ in range(NB)]
    zrow_v = rest.pop(0)
    agg_sh = rest.pop(0)
    gsems = [rest.pop(0) for _ in range(NB)]
    ssems = [rest.pop(0) for _ in range(NB)]
    if with_deg:
      ones_v, zdeg_v, deg_sh = rest
    core = lax.axis_index("c")
    sid = lax.axis_index("s")

    def fill(i, _):
      for j in range(dw // 16):
        zrow_v[i, pl.ds(j * 16, 16)] = jnp.zeros((16,), f32)
      return 0

    lax.fori_loop(0, ZR, fill, 0)
    _zero_shared(sid, zrow_v, agg_sh)
    if with_deg:
      def dfill(i, _):
        zdeg_v[i, :] = jnp.zeros((16,), f32)
        return 0

      lax.fori_loop(0, ZR, dfill, 0)

      def ofill(i, _):
        ones_v[i, :] = jnp.full((16,), 0.5, f32)
        return 0

      lax.fori_loop(0, K, ofill, 0)
      _zero_shared(sid, zdeg_v, deg_sh)
    pltpu.sync_copy(ei_hbm.at[0, sid], srcs_v)
    pltpu.sync_copy(ei_hbm.at[1, sid], dsts_v)
    if interleaved:
      def xform(i, _):
        sl = pl.ds(i * 16, 16)
        srcs_v[sl] = srcs_v[sl] * 2 + core
        return 0

      lax.fori_loop(0, EPW // 16, xform, 0)
    plsc.subcore_barrier()

    feat = feat_hbm if interleaved else feat_hbm.at[core]

    def sidx(c):
      return srcs_v.at[pl.ds(c * K, K)]

    def didx(c):
      return dsts_v.at[pl.ds(c * K, K)]

    for b in range(NB - 1):
      pltpu.async_copy(feat.at[sidx(b)], rb[b], gsems[b])

    def ring(i, _):
      for b in range(NB):
        c = i * NB + b
        nxt = (b + NB - 1) % NB
        pltpu.make_async_copy(feat.at[sidx(c)], rb[b], gsems[b]).wait()

        @pl.when(c + NB - 1 < nch)
        def _():
          @pl.when(c >= 1)
          def _():
            pltpu.make_async_copy(rb[nxt], agg_sh.at[didx(c)],
                                  ssems[nxt]).wait()
          pltpu.async_copy(feat.at[sidx(c + NB - 1)], rb[nxt], gsems[nxt])

        pltpu.async_copy(rb[b], agg_sh.at[didx(c)], ssems[b], add=True)
        if with_deg:
          pltpu.sync_copy(ones_v, deg_sh.at[didx(c)], add=True)
      return 0

    lax.fori_loop(0, nch // NB, ring, 0)
    tb = nch % NB
    pltpu.make_async_copy(rb[tb], agg_sh.at[didx(0)], ssems[tb]).wait()
    tsrc = srcs_v.at[pl.ds(NCH_A * K, KT)]
    tdst = dsts_v.at[pl.ds(NCH_A * K, KT)]
    pltpu.async_copy(feat.at[tsrc], rb[tb].at[pl.ds(0, KT)], gsems[tb])
    pltpu.make_async_copy(feat.at[tsrc], rb[tb].at[pl.ds(0, KT)],
                          gsems[tb]).wait()
    pltpu.async_copy(rb[tb].at[pl.ds(0, KT)], agg_sh.at[tdst], ssems[tb],
                     add=True)
    if with_deg:
      pltpu.sync_copy(ones_v.at[pl.ds(0, KT)], deg_sh.at[tdst], add=True)
    for b in range(NB):
      if b == tb:
        pltpu.make_async_copy(rb[b].at[pl.ds(0, KT)], agg_sh.at[didx(0)],
                              ssems[b]).wait()
      else:
        pltpu.make_async_copy(rb[b], agg_sh.at[didx(0)], ssems[b]).wait()
    plsc.subcore_barrier()
    _writeback(core, sid, agg_sh, agg_hbm)
    if with_deg:
      _writeback(core, sid, deg_sh, deg_hbm)

  return functools.partial(
      pl.kernel, mesh=mesh, out_type=out_type, scratch_types=scratch,
      compiler_params=pltpu.CompilerParams(
          use_tc_tiling_on_sc=False, skip_device_barrier=True))(body)


_make_agg = functools.lru_cache(maxsize=None)(_make_agg)

BN = 2000


def _make_combine(relu, logsm, split_x, split_out, agg_full):
  def body(agg_ref, deg_ref, x_ref, wn_ref, ws_ref, b_ref, out_ref):
    if agg_full:
      agg = agg_ref[0] + agg_ref[1]
    else:
      agg = jnp.concatenate([agg_ref[0], agg_ref[1]], axis=1)
    deg = deg_ref[0][:, 0:1] + deg_ref[1][:, 0:1]
    mean = agg / jnp.maximum(deg, 1.0)
    if split_x:
      xv = jnp.concatenate([x_ref[0], x_ref[1]], axis=1)
    else:
      xv = x_ref[...]
    h = (jnp.dot(mean, wn_ref[...], preferred_element_type=f32,
                 precision=lax.Precision.HIGHEST)
         + jnp.dot(xv, ws_ref[...], preferred_element_type=f32,
                   precision=lax.Precision.HIGHEST)
         + b_ref[...])
    if relu:
      h = jnp.maximum(h, 0.0)
    if logsm:
      m = jnp.max(h, axis=1, keepdims=True)
      h = h - (jnp.log(jnp.sum(jnp.exp(h - m), axis=1, keepdims=True)) + m)
    if split_out:
      out_ref[0] = h[:, :D2]
      out_ref[1] = h[:, D2:]
    else:
      out_ref[...] = h

  x_spec = (pl.BlockSpec((2, BN, D2), lambda i: (0, i, 0)) if split_x
            else pl.BlockSpec((BN, D), lambda i: (i, 0)))
  if split_out:
    out_spec = pl.BlockSpec((2, BN, D2), lambda i: (0, i, 0))
    out_shape = jax.ShapeDtypeStruct((NC, N, D2), f32)
  else:
    out_spec = pl.BlockSpec((BN, D), lambda i: (i, 0))
    out_shape = jax.ShapeDtypeStruct((N, D), f32)

  return pl.pallas_call(
      body,
      grid=(N // BN,),
      in_specs=[
          pl.BlockSpec((2, BN, D if agg_full else D2), lambda i: (0, i, 0)),
          pl.BlockSpec((2, BN, DW), lambda i: (0, i, 0)),
          x_spec,
          pl.BlockSpec((D, D), lambda i: (0, 0)),
          pl.BlockSpec((D, D), lambda i: (0, 0)),
          pl.BlockSpec((1, D), lambda i: (0, 0)),
      ],
      out_specs=out_spec,
      out_shape=out_shape,
  )


_combine1 = _make_combine(relu=True, logsm=False, split_x=False,
                          split_out=True, agg_full=False)
_combine2 = _make_combine(relu=False, logsm=True, split_x=True,
                          split_out=False, agg_full=False)


def kernel(x, edge_index, W1_neigh, W1_self, b1, W2_neigh, W2_self, b2):
  ei4 = edge_index.reshape(2, NS, EPW)
  xi = x.reshape(NC * N, D2)
  agg1, degp = _make_agg(True, True)(xi, ei4)
  h2 = _combine1(agg1, degp, x, W1_neigh, W1_self, b1.reshape(1, D))
  (agg2,) = _make_agg(False, False)(h2, ei4)
  return _combine2(agg2, degp, h2, W2_neigh, W2_self, b2.reshape(1, D))
